# Initial kernel scaffold; baseline (speedup 1.0000x reference)
import jax
import jax.numpy as jnp
from jax.experimental import pallas as pl


def kernel(X, A, W1, b1, W2, b2, Wmu, bmu):
    raise NotImplementedError("write your pallas kernel here")


# speedup vs baseline: 8.7993x; 8.7993x over previous
import functools

import jax
import jax.numpy as jnp
from jax import lax
from jax.experimental import pallas as pl
from jax.experimental.pallas import tpu as pltpu
from jax.experimental.pallas import tpu_sc as plsc

N = 10000
E = 160000
IN_CH = 128
H1 = 256
H2 = 128
LATENT = 64

NC = 2
NS = 16
K = 125
NB = E // (NC * NS * K)
SLAB = 624
TAIL = N - NS * SLAB
ZR = 104
LANES = 16

HIGHEST = lax.Precision.HIGHEST


def _fill(ref, nrows, ncol, value):
    def body(r, _):
        for f in range(ncol // LANES):
            ref[r, pl.ds(f * LANES, LANES)] = jnp.full(
                (LANES,), value, jnp.float32)
        return 0
    lax.fori_loop(0, nrows, body, 0)


def _make_deg():
    mesh = plsc.VectorSubcoreMesh(core_axis_name="c", subcore_axis_name="s")

    @functools.partial(
        pl.kernel,
        out_type=jax.ShapeDtypeStruct((NC, N, LANES), jnp.float32),
        mesh=mesh,
        scratch_types=[
            pltpu.VMEM((NB, K), jnp.int32),
            pltpu.VMEM((K, LANES), jnp.float32),
            pltpu.VMEM((ZR, LANES), jnp.float32),
            pltpu.VMEM_SHARED((N, LANES), jnp.float32),
        ],
    )
    def deg(dst_hbm, cnt_hbm, dst_v, ones_v, zeros_v, acc_sh):
        c = lax.axis_index("c")
        s = lax.axis_index("s")
        pltpu.sync_copy(dst_hbm.at[c, s], dst_v)
        _fill(ones_v, K, LANES, 1.0)
        _fill(zeros_v, ZR, LANES, 0.0)
        row0 = pl.multiple_of(s * SLAB, 8)
        for r in range(SLAB // ZR):
            pltpu.sync_copy(zeros_v,
                            acc_sh.at[pl.ds(pl.multiple_of(row0 + r * ZR, 8),
                                            ZR)])
        @pl.when(s == 0)
        def _():
            pltpu.sync_copy(zeros_v.at[pl.ds(0, TAIL)],
                            acc_sh.at[pl.ds(NS * SLAB, TAIL)])
        plsc.subcore_barrier()
        def body(j, _):
            pltpu.sync_copy(ones_v, acc_sh.at[dst_v.at[j]], add=True)
            return 0
        lax.fori_loop(0, NB, body, 0)
        plsc.subcore_barrier()
        pltpu.sync_copy(acc_sh.at[pl.ds(row0, SLAB)],
                        cnt_hbm.at[c, pl.ds(row0, SLAB)])
        @pl.when(s == 0)
        def _():
            pltpu.sync_copy(acc_sh.at[pl.ds(NS * SLAB, TAIL)],
                            cnt_hbm.at[c, pl.ds(NS * SLAB, TAIL)])

    return deg


def _make_scatter(F):
    mesh = plsc.VectorSubcoreMesh(core_axis_name="c", subcore_axis_name="s")

    @functools.partial(
        pl.kernel,
        out_type=jax.ShapeDtypeStruct((NC, N, F), jnp.float32),
        mesh=mesh,
        scratch_types=[
            pltpu.VMEM((NB, K), jnp.int32),
            pltpu.VMEM((NB, K), jnp.int32),
            pltpu.VMEM((2, K, F), jnp.float32),
            pltpu.VMEM_SHARED((N, F), jnp.float32),
            pltpu.SemaphoreType.DMA,
            pltpu.SemaphoreType.DMA,
        ],
    )
    def scat(table_hbm, src_hbm, dst_hbm, out_hbm,
             src_v, dst_v, rows_v, acc_sh, sem0, sem1):
        c = lax.axis_index("c")
        s = lax.axis_index("s")
        pltpu.sync_copy(src_hbm.at[c, s], src_v)
        pltpu.sync_copy(dst_hbm.at[c, s], dst_v)
        _fill(rows_v.at[0], ZR, F, 0.0)
        row0 = pl.multiple_of(s * SLAB, 8)
        for r in range(SLAB // ZR):
            pltpu.sync_copy(rows_v.at[0, pl.ds(0, ZR)],
                            acc_sh.at[pl.ds(pl.multiple_of(row0 + r * ZR, 8),
                                            ZR)])
        @pl.when(s == 0)
        def _():
            pltpu.sync_copy(rows_v.at[0, pl.ds(0, TAIL)],
                            acc_sh.at[pl.ds(NS * SLAB, TAIL)])
        plsc.subcore_barrier()
        sems = (sem0, sem1)
        desc = [None, None]
        desc[0] = pltpu.async_copy(table_hbm.at[src_v.at[0]],
                                   rows_v.at[0], sem0)
        for j in range(NB):
            b = j % 2
            desc[b].wait()
            if j + 1 < NB:
                nb_ = (j + 1) % 2
                desc[nb_] = pltpu.async_copy(
                    table_hbm.at[src_v.at[j + 1]], rows_v.at[nb_], sems[nb_])
            pltpu.sync_copy(rows_v.at[b], acc_sh.at[dst_v.at[j]], add=True)
        plsc.subcore_barrier()
        pltpu.sync_copy(acc_sh.at[pl.ds(row0, SLAB)],
                        out_hbm.at[c, pl.ds(row0, SLAB)])
        @pl.when(s == 0)
        def _():
            pltpu.sync_copy(acc_sh.at[pl.ds(NS * SLAB, TAIL)],
                            out_hbm.at[c, pl.ds(NS * SLAB, TAIL)])

    return scat


_deg_call = _make_deg()
_scat128 = _make_scatter(128)


RB = 1000


def _dinv_body(cnt_ref, o_ref):
    deg = 1.0 + cnt_ref[0, :, 0:1] + cnt_ref[1, :, 0:1]
    o_ref[...] = lax.rsqrt(deg)


def _dinv(cnt):
    return pl.pallas_call(
        _dinv_body,
        grid=(N // RB,),
        in_specs=[pl.BlockSpec((NC, RB, LANES), lambda i: (0, i, 0))],
        out_specs=pl.BlockSpec((RB, 1), lambda i: (i, 0)),
        out_shape=jax.ShapeDtypeStruct((N, 1), jnp.float32),
    )(cnt)


def _mm1_body(x_ref, w_ref, dinv_ref, o_ref):
    h = lax.dot(x_ref[...], w_ref[...], precision=HIGHEST,
                preferred_element_type=jnp.float32)
    o_ref[...] = dinv_ref[...] * h


def _mm1(X, W1, dinv):
    return pl.pallas_call(
        _mm1_body,
        grid=(N // RB,),
        in_specs=[
            pl.BlockSpec((RB, IN_CH), lambda i: (i, 0)),
            pl.BlockSpec((IN_CH, H1), lambda i: (0, 0)),
            pl.BlockSpec((RB, 1), lambda i: (i, 0)),
        ],
        out_specs=pl.BlockSpec((RB, H1), lambda i: (i, 0)),
        out_shape=jax.ShapeDtypeStruct((N, H1), jnp.float32),
    )(X, W1, dinv)


def _comb_body(s_ref, hp_ref, b_ref, dinv_ref, w_ref, o_ref):
    dinv = dinv_ref[...]
    u = dinv * (s_ref[0] + s_ref[1] + hp_ref[...]) + b_ref[...]
    a = jnp.where(u >= 0, u, 0.2 * u)
    o_ref[...] = dinv * lax.dot(a, w_ref[...], precision=HIGHEST,
                                preferred_element_type=jnp.float32)


def _comb(s, hp, b, dinv, W, F, Fn):
    return pl.pallas_call(
        _comb_body,
        grid=(N // RB,),
        in_specs=[
            pl.BlockSpec((NC, RB, F), lambda i: (0, i, 0)),
            pl.BlockSpec((RB, F), lambda i: (i, 0)),
            pl.BlockSpec((1, F), lambda i: (0, 0)),
            pl.BlockSpec((RB, 1), lambda i: (i, 0)),
            pl.BlockSpec((F, Fn), lambda i: (0, 0)),
        ],
        out_specs=pl.BlockSpec((RB, Fn), lambda i: (i, 0)),
        out_shape=jax.ShapeDtypeStruct((N, Fn), jnp.float32),
    )(s, hp, b.reshape(1, F), dinv, W)


def _zfin_body(s_ref, hp_ref, b_ref, dinv_ref, o_ref):
    o_ref[...] = (dinv_ref[...] * (s_ref[0] + s_ref[1] + hp_ref[...])
                  + b_ref[...])


def _zfin(s, hp, b, dinv):
    return pl.pallas_call(
        _zfin_body,
        grid=(N // RB,),
        in_specs=[
            pl.BlockSpec((NC, RB, LATENT), lambda i: (0, i, 0)),
            pl.BlockSpec((RB, LATENT), lambda i: (i, 0)),
            pl.BlockSpec((1, LATENT), lambda i: (0, 0)),
            pl.BlockSpec((RB, 1), lambda i: (i, 0)),
        ],
        out_specs=pl.BlockSpec((RB, LATENT), lambda i: (i, 0)),
        out_shape=jax.ShapeDtypeStruct((N, LATENT), jnp.float32),
    )(s, hp, b.reshape(1, LATENT), dinv)


def _dec_body(za_ref, zt_ref, o_ref):
    p = lax.dot(za_ref[...], zt_ref[...], precision=HIGHEST,
                preferred_element_type=jnp.float32)
    o_ref[...] = jax.nn.sigmoid(p)


CB = 1024


def _dec(z, zT):
    return pl.pallas_call(
        _dec_body,
        grid=(N // RB, pl.cdiv(N, CB)),
        in_specs=[
            pl.BlockSpec((RB, LATENT), lambda i, j: (i, 0)),
            pl.BlockSpec((LATENT, CB), lambda i, j: (0, j)),
        ],
        out_specs=pl.BlockSpec((RB, CB), lambda i, j: (i, j)),
        out_shape=jax.ShapeDtypeStruct((N, N), jnp.float32),
    )(z, zT)


def kernel(X, A, W1, b1, W2, b2, Wmu, bmu):
    A32 = A.astype(jnp.int32)
    src = A32[0].reshape(NC, NS, NB, K)
    dst = A32[1].reshape(NC, NS, NB, K)

    cnt = _deg_call(dst)
    dinv = _dinv(cnt)

    h1p = _mm1(X, W1, dinv)
    sa = _scat128(h1p[:, :128], src, dst)
    sb = _scat128(h1p[:, 128:], src, dst)
    s1 = jnp.concatenate([sa, sb], axis=2)

    h2p = _comb(s1, h1p, b1, dinv, W2, H1, H2)
    s2 = _scat128(h2p, src, dst)

    h3p = _comb(s2, h2p, b2, dinv, Wmu, H2, LATENT)
    h3p_pad = jnp.concatenate(
        [h3p, jnp.zeros((N, 128 - LATENT), jnp.float32)], axis=1)
    s3 = _scat128(h3p_pad, src, dst)[:, :, :LATENT]
    z = _zfin(s3, h3p, bmu, dinv)

    return _dec(z, z.T)


# speedup vs baseline: 9.9972x; 1.1361x over previous
import functools

import jax
import jax.numpy as jnp
from jax import lax
from jax.experimental import pallas as pl
from jax.experimental.pallas import tpu as pltpu
from jax.experimental.pallas import tpu_sc as plsc

N = 10000
E = 160000
IN_CH = 128
H1 = 256
H2 = 128
LATENT = 64

NC = 2
NS = 16
K = 125
NB = E // (NC * NS * K)
SLAB = 624
TAIL = N - NS * SLAB
ZR = 104
LANES = 16

HIGHEST = lax.Precision.HIGHEST


def _fill(ref, nrows, ncol, value):
    def body(r, _):
        for f in range(ncol // LANES):
            ref[r, pl.ds(f * LANES, LANES)] = jnp.full(
                (LANES,), value, jnp.float32)
        return 0
    lax.fori_loop(0, nrows, body, 0)


def _make_deg():
    mesh = plsc.VectorSubcoreMesh(core_axis_name="c", subcore_axis_name="s")

    @functools.partial(
        pl.kernel,
        out_type=jax.ShapeDtypeStruct((NC, N, LANES), jnp.float32),
        mesh=mesh,
        scratch_types=[
            pltpu.VMEM((NB, K), jnp.int32),
            pltpu.VMEM((K, LANES), jnp.float32),
            pltpu.VMEM((ZR, LANES), jnp.float32),
            pltpu.VMEM_SHARED((N, LANES), jnp.float32),
        ],
    )
    def deg(dst_hbm, cnt_hbm, dst_v, ones_v, zeros_v, acc_sh):
        c = lax.axis_index("c")
        s = lax.axis_index("s")
        pltpu.sync_copy(dst_hbm.at[c, s], dst_v)
        _fill(ones_v, K, LANES, 1.0)
        _fill(zeros_v, ZR, LANES, 0.0)
        row0 = pl.multiple_of(s * SLAB, 8)
        for r in range(SLAB // ZR):
            pltpu.sync_copy(zeros_v,
                            acc_sh.at[pl.ds(pl.multiple_of(row0 + r * ZR, 8),
                                            ZR)])
        @pl.when(s == 0)
        def _():
            pltpu.sync_copy(zeros_v.at[pl.ds(0, TAIL)],
                            acc_sh.at[pl.ds(NS * SLAB, TAIL)])
        plsc.subcore_barrier()
        def body(j, _):
            pltpu.sync_copy(ones_v, acc_sh.at[dst_v.at[j]], add=True)
            return 0
        lax.fori_loop(0, NB, body, 0)
        plsc.subcore_barrier()
        pltpu.sync_copy(acc_sh.at[pl.ds(row0, SLAB)],
                        cnt_hbm.at[c, pl.ds(row0, SLAB)])
        @pl.when(s == 0)
        def _():
            pltpu.sync_copy(acc_sh.at[pl.ds(NS * SLAB, TAIL)],
                            cnt_hbm.at[c, pl.ds(NS * SLAB, TAIL)])

    return deg


def _make_scatter(F):
    mesh = plsc.VectorSubcoreMesh(core_axis_name="c", subcore_axis_name="s")

    @functools.partial(
        pl.kernel,
        out_type=jax.ShapeDtypeStruct((NC, N, F), jnp.float32),
        mesh=mesh,
        scratch_types=[
            pltpu.VMEM((NB, K), jnp.int32),
            pltpu.VMEM((NB, K), jnp.int32),
            pltpu.VMEM((2, K, F), jnp.float32),
            pltpu.VMEM_SHARED((N, F), jnp.float32),
            pltpu.SemaphoreType.DMA,
            pltpu.SemaphoreType.DMA,
        ],
    )
    def scat(table_hbm, src_hbm, dst_hbm, out_hbm,
             src_v, dst_v, rows_v, acc_sh, sem0, sem1):
        c = lax.axis_index("c")
        s = lax.axis_index("s")
        pltpu.sync_copy(src_hbm.at[c, s], src_v)
        pltpu.sync_copy(dst_hbm.at[c, s], dst_v)
        _fill(rows_v.at[0], ZR, F, 0.0)
        row0 = pl.multiple_of(s * SLAB, 8)
        for r in range(SLAB // ZR):
            pltpu.sync_copy(rows_v.at[0, pl.ds(0, ZR)],
                            acc_sh.at[pl.ds(pl.multiple_of(row0 + r * ZR, 8),
                                            ZR)])
        @pl.when(s == 0)
        def _():
            pltpu.sync_copy(rows_v.at[0, pl.ds(0, TAIL)],
                            acc_sh.at[pl.ds(NS * SLAB, TAIL)])
        plsc.subcore_barrier()
        sems = (sem0, sem1)
        desc = [None, None]
        desc[0] = pltpu.async_copy(table_hbm.at[src_v.at[0]],
                                   rows_v.at[0], sem0)
        for j in range(NB):
            b = j % 2
            desc[b].wait()
            if j + 1 < NB:
                nb_ = (j + 1) % 2
                desc[nb_] = pltpu.async_copy(
                    table_hbm.at[src_v.at[j + 1]], rows_v.at[nb_], sems[nb_])
            pltpu.sync_copy(rows_v.at[b], acc_sh.at[dst_v.at[j]], add=True)
        plsc.subcore_barrier()
        pltpu.sync_copy(acc_sh.at[pl.ds(row0, SLAB)],
                        out_hbm.at[c, pl.ds(row0, SLAB)])
        @pl.when(s == 0)
        def _():
            pltpu.sync_copy(acc_sh.at[pl.ds(NS * SLAB, TAIL)],
                            out_hbm.at[c, pl.ds(NS * SLAB, TAIL)])

    return scat


_deg_call = _make_deg()
_scat128 = _make_scatter(128)


RB = 1000


def _pre_body(cnt_ref, x_ref, dinv_ref, xp_ref):
    deg = 1.0 + cnt_ref[0, :, 0:1] + cnt_ref[1, :, 0:1]
    dinv = lax.rsqrt(deg)
    dinv_ref[...] = dinv
    xp_ref[...] = dinv * x_ref[...]


def _pre(cnt, X):
    return pl.pallas_call(
        _pre_body,
        grid=(N // RB,),
        in_specs=[
            pl.BlockSpec((NC, RB, LANES), lambda i: (0, i, 0)),
            pl.BlockSpec((RB, IN_CH), lambda i: (i, 0)),
        ],
        out_specs=[
            pl.BlockSpec((RB, 1), lambda i: (i, 0)),
            pl.BlockSpec((RB, IN_CH), lambda i: (i, 0)),
        ],
        out_shape=[
            jax.ShapeDtypeStruct((N, 1), jnp.float32),
            jax.ShapeDtypeStruct((N, IN_CH), jnp.float32),
        ],
    )(cnt, X)


def _l1_body(s_ref, xp_ref, w1_ref, b1_ref, w2_ref, dinv_ref, o_ref):
    dinv = dinv_ref[...]
    agg = s_ref[0] + s_ref[1] + xp_ref[...]
    u = dinv * lax.dot(agg, w1_ref[...], precision=HIGHEST,
                       preferred_element_type=jnp.float32) + b1_ref[...]
    a = jnp.where(u >= 0, u, 0.2 * u)
    o_ref[...] = dinv * lax.dot(a, w2_ref[...], precision=HIGHEST,
                                preferred_element_type=jnp.float32)


def _l1(s1, Xp, W1, b1, W2, dinv):
    return pl.pallas_call(
        _l1_body,
        grid=(N // RB,),
        in_specs=[
            pl.BlockSpec((NC, RB, IN_CH), lambda i: (0, i, 0)),
            pl.BlockSpec((RB, IN_CH), lambda i: (i, 0)),
            pl.BlockSpec((IN_CH, H1), lambda i: (0, 0)),
            pl.BlockSpec((1, H1), lambda i: (0, 0)),
            pl.BlockSpec((H1, H2), lambda i: (0, 0)),
            pl.BlockSpec((RB, 1), lambda i: (i, 0)),
        ],
        out_specs=pl.BlockSpec((RB, H2), lambda i: (i, 0)),
        out_shape=jax.ShapeDtypeStruct((N, H2), jnp.float32),
    )(s1, Xp, W1, b1.reshape(1, H1), W2, dinv)


def _l2_body(s_ref, hp_ref, b_ref, dinv_ref, o_ref):
    dinv = dinv_ref[...]
    u = dinv * (s_ref[0] + s_ref[1] + hp_ref[...]) + b_ref[...]
    a = jnp.where(u >= 0, u, 0.2 * u)
    o_ref[...] = dinv * a


def _l2(s, hp, b, dinv):
    return pl.pallas_call(
        _l2_body,
        grid=(N // RB,),
        in_specs=[
            pl.BlockSpec((NC, RB, H2), lambda i: (0, i, 0)),
            pl.BlockSpec((RB, H2), lambda i: (i, 0)),
            pl.BlockSpec((1, H2), lambda i: (0, 0)),
            pl.BlockSpec((RB, 1), lambda i: (i, 0)),
        ],
        out_specs=pl.BlockSpec((RB, H2), lambda i: (i, 0)),
        out_shape=jax.ShapeDtypeStruct((N, H2), jnp.float32),
    )(s, hp, b.reshape(1, H2), dinv)


def _zfin_body(s_ref, ap_ref, b_ref, dinv_ref, w_ref, o_ref):
    agg = s_ref[0] + s_ref[1] + ap_ref[...]
    o_ref[...] = dinv_ref[...] * lax.dot(
        agg, w_ref[...], precision=HIGHEST,
        preferred_element_type=jnp.float32) + b_ref[...]


def _zfin(s, ap, b, dinv, W):
    return pl.pallas_call(
        _zfin_body,
        grid=(N // RB,),
        in_specs=[
            pl.BlockSpec((NC, RB, H2), lambda i: (0, i, 0)),
            pl.BlockSpec((RB, H2), lambda i: (i, 0)),
            pl.BlockSpec((1, LATENT), lambda i: (0, 0)),
            pl.BlockSpec((RB, 1), lambda i: (i, 0)),
            pl.BlockSpec((H2, LATENT), lambda i: (0, 0)),
        ],
        out_specs=pl.BlockSpec((RB, LATENT), lambda i: (i, 0)),
        out_shape=jax.ShapeDtypeStruct((N, LATENT), jnp.float32),
    )(s, ap, b.reshape(1, LATENT), dinv, W)


def _dec_body(za_ref, zt_ref, o_ref):
    p = lax.dot(za_ref[...], zt_ref[...], precision=HIGHEST,
                preferred_element_type=jnp.float32)
    o_ref[...] = jax.nn.sigmoid(p)


CB = 1024


def _dec(z, zT):
    return pl.pallas_call(
        _dec_body,
        grid=(N // RB, pl.cdiv(N, CB)),
        in_specs=[
            pl.BlockSpec((RB, LATENT), lambda i, j: (i, 0)),
            pl.BlockSpec((LATENT, CB), lambda i, j: (0, j)),
        ],
        out_specs=pl.BlockSpec((RB, CB), lambda i, j: (i, j)),
        out_shape=jax.ShapeDtypeStruct((N, N), jnp.float32),
    )(z, zT)


def kernel(X, A, W1, b1, W2, b2, Wmu, bmu):
    A32 = A.astype(jnp.int32)
    src = A32[0].reshape(NC, NS, NB, K)
    dst = A32[1].reshape(NC, NS, NB, K)

    cnt = _deg_call(dst)
    dinv, Xp = _pre(cnt, X)

    s1 = _scat128(Xp, src, dst)
    h2p = _l1(s1, Xp, W1, b1, W2, dinv)
    s2 = _scat128(h2p, src, dst)

    a2p = _l2(s2, h2p, b2, dinv)
    s3 = _scat128(a2p, src, dst)
    z = _zfin(s3, a2p, bmu, dinv, Wmu)

    return _dec(z, z.T)


# speedup vs baseline: 14.6611x; 1.4665x over previous
import functools

import jax
import jax.numpy as jnp
from jax import lax
from jax.experimental import pallas as pl
from jax.experimental.pallas import tpu as pltpu
from jax.experimental.pallas import tpu_sc as plsc

N = 10000
E = 160000
IN_CH = 128
H1 = 256
H2 = 128
LATENT = 64

NC = 2
NS = 16
K = 125
NB = E // (NC * NS * K)
SLAB = 624
TAIL = N - NS * SLAB
ZR = 104
LANES = 16

HIGHEST = lax.Precision.HIGHEST


def _fill(ref, nrows, ncol, value):
    def body(r, _):
        for f in range(ncol // LANES):
            ref[r, pl.ds(f * LANES, LANES)] = jnp.full(
                (LANES,), value, jnp.float32)
        return 0
    lax.fori_loop(0, nrows, body, 0)


def _make_deg():
    mesh = plsc.VectorSubcoreMesh(core_axis_name="c", subcore_axis_name="s")

    @functools.partial(
        pl.kernel,
        out_type=jax.ShapeDtypeStruct((NC, N, LANES), jnp.float32),
        mesh=mesh,
        scratch_types=[
            pltpu.VMEM((NB, K), jnp.int32),
            pltpu.VMEM((K, LANES), jnp.float32),
            pltpu.VMEM((ZR, LANES), jnp.float32),
            pltpu.VMEM_SHARED((N, LANES), jnp.float32),
        ],
    )
    def deg(dst_hbm, cnt_hbm, dst_v, ones_v, zeros_v, acc_sh):
        c = lax.axis_index("c")
        s = lax.axis_index("s")
        pltpu.sync_copy(dst_hbm.at[c, s], dst_v)
        _fill(ones_v, K, LANES, 1.0)
        _fill(zeros_v, ZR, LANES, 0.0)
        row0 = pl.multiple_of(s * SLAB, 8)
        for r in range(SLAB // ZR):
            pltpu.sync_copy(zeros_v,
                            acc_sh.at[pl.ds(pl.multiple_of(row0 + r * ZR, 8),
                                            ZR)])
        @pl.when(s == 0)
        def _():
            pltpu.sync_copy(zeros_v.at[pl.ds(0, TAIL)],
                            acc_sh.at[pl.ds(NS * SLAB, TAIL)])
        plsc.subcore_barrier()
        def body(j, _):
            pltpu.sync_copy(ones_v, acc_sh.at[dst_v.at[j]], add=True)
            return 0
        lax.fori_loop(0, NB, body, 0)
        plsc.subcore_barrier()
        pltpu.sync_copy(acc_sh.at[pl.ds(row0, SLAB)],
                        cnt_hbm.at[c, pl.ds(row0, SLAB)])
        @pl.when(s == 0)
        def _():
            pltpu.sync_copy(acc_sh.at[pl.ds(NS * SLAB, TAIL)],
                            cnt_hbm.at[c, pl.ds(NS * SLAB, TAIL)])

    return deg


def _make_scatter(F):
    mesh = plsc.VectorSubcoreMesh(core_axis_name="c", subcore_axis_name="s")

    @functools.partial(
        pl.kernel,
        out_type=jax.ShapeDtypeStruct((NC, N, F), jnp.float32),
        mesh=mesh,
        scratch_types=[
            pltpu.VMEM((NB, K), jnp.int32),
            pltpu.VMEM((NB, K), jnp.int32),
            pltpu.VMEM((2, K, F), jnp.float32),
            pltpu.VMEM_SHARED((N, F), jnp.float32),
            pltpu.SemaphoreType.DMA,
            pltpu.SemaphoreType.DMA,
        ],
    )
    def scat(table_hbm, src_hbm, dst_hbm, out_hbm,
             src_v, dst_v, rows_v, acc_sh, sem0, sem1):
        c = lax.axis_index("c")
        s = lax.axis_index("s")
        pltpu.sync_copy(src_hbm.at[c, s], src_v)
        pltpu.sync_copy(dst_hbm.at[c, s], dst_v)
        _fill(rows_v.at[0], ZR, F, 0.0)
        row0 = pl.multiple_of(s * SLAB, 8)
        for r in range(SLAB // ZR):
            pltpu.sync_copy(rows_v.at[0, pl.ds(0, ZR)],
                            acc_sh.at[pl.ds(pl.multiple_of(row0 + r * ZR, 8),
                                            ZR)])
        @pl.when(s == 0)
        def _():
            pltpu.sync_copy(rows_v.at[0, pl.ds(0, TAIL)],
                            acc_sh.at[pl.ds(NS * SLAB, TAIL)])
        plsc.subcore_barrier()
        sems = (sem0, sem1)
        desc = [None, None]
        desc[0] = pltpu.async_copy(table_hbm.at[src_v.at[0]],
                                   rows_v.at[0], sem0)
        for j in range(NB):
            b = j % 2
            desc[b].wait()
            if j + 1 < NB:
                nb_ = (j + 1) % 2
                desc[nb_] = pltpu.async_copy(
                    table_hbm.at[src_v.at[j + 1]], rows_v.at[nb_], sems[nb_])
            pltpu.sync_copy(rows_v.at[b], acc_sh.at[dst_v.at[j]], add=True)
        plsc.subcore_barrier()
        pltpu.sync_copy(acc_sh.at[pl.ds(row0, SLAB)],
                        out_hbm.at[c, pl.ds(row0, SLAB)])
        @pl.when(s == 0)
        def _():
            pltpu.sync_copy(acc_sh.at[pl.ds(NS * SLAB, TAIL)],
                            out_hbm.at[c, pl.ds(NS * SLAB, TAIL)])

    return scat


_deg_call = _make_deg()
_scat128 = _make_scatter(128)


RB = 1000


def _pre_body(cnt_ref, x_ref, dinv_ref, xp_ref):
    deg = 1.0 + cnt_ref[0, :, 0:1] + cnt_ref[1, :, 0:1]
    dinv = lax.rsqrt(deg)
    dinv_ref[...] = dinv
    xp_ref[...] = dinv * x_ref[...]


def _pre(cnt, X):
    return pl.pallas_call(
        _pre_body,
        grid=(N // RB,),
        in_specs=[
            pl.BlockSpec((NC, RB, LANES), lambda i: (0, i, 0)),
            pl.BlockSpec((RB, IN_CH), lambda i: (i, 0)),
        ],
        out_specs=[
            pl.BlockSpec((RB, 1), lambda i: (i, 0)),
            pl.BlockSpec((RB, IN_CH), lambda i: (i, 0)),
        ],
        out_shape=[
            jax.ShapeDtypeStruct((N, 1), jnp.float32),
            jax.ShapeDtypeStruct((N, IN_CH), jnp.float32),
        ],
    )(cnt, X)


def _l1_body(s_ref, xp_ref, w1_ref, b1_ref, w2_ref, dinv_ref, o_ref):
    dinv = dinv_ref[...]
    agg = s_ref[0] + s_ref[1] + xp_ref[...]
    u = dinv * lax.dot(agg, w1_ref[...], precision=HIGHEST,
                       preferred_element_type=jnp.float32) + b1_ref[...]
    a = jnp.where(u >= 0, u, 0.2 * u)
    o_ref[...] = dinv * lax.dot(a, w2_ref[...], precision=HIGHEST,
                                preferred_element_type=jnp.float32)


def _l1(s1, Xp, W1, b1, W2, dinv):
    return pl.pallas_call(
        _l1_body,
        grid=(N // RB,),
        in_specs=[
            pl.BlockSpec((NC, RB, IN_CH), lambda i: (0, i, 0)),
            pl.BlockSpec((RB, IN_CH), lambda i: (i, 0)),
            pl.BlockSpec((IN_CH, H1), lambda i: (0, 0)),
            pl.BlockSpec((1, H1), lambda i: (0, 0)),
            pl.BlockSpec((H1, H2), lambda i: (0, 0)),
            pl.BlockSpec((RB, 1), lambda i: (i, 0)),
        ],
        out_specs=pl.BlockSpec((RB, H2), lambda i: (i, 0)),
        out_shape=jax.ShapeDtypeStruct((N, H2), jnp.float32),
    )(s1, Xp, W1, b1.reshape(1, H1), W2, dinv)


def _l2_body(s_ref, hp_ref, b_ref, dinv_ref, o_ref):
    dinv = dinv_ref[...]
    u = dinv * (s_ref[0] + s_ref[1] + hp_ref[...]) + b_ref[...]
    a = jnp.where(u >= 0, u, 0.2 * u)
    o_ref[...] = dinv * a


def _l2(s, hp, b, dinv):
    return pl.pallas_call(
        _l2_body,
        grid=(N // RB,),
        in_specs=[
            pl.BlockSpec((NC, RB, H2), lambda i: (0, i, 0)),
            pl.BlockSpec((RB, H2), lambda i: (i, 0)),
            pl.BlockSpec((1, H2), lambda i: (0, 0)),
            pl.BlockSpec((RB, 1), lambda i: (i, 0)),
        ],
        out_specs=pl.BlockSpec((RB, H2), lambda i: (i, 0)),
        out_shape=jax.ShapeDtypeStruct((N, H2), jnp.float32),
    )(s, hp, b.reshape(1, H2), dinv)


LS = 3 * LATENT


def _zfin_body(s_ref, ap_ref, b_ref, dinv_ref, w_ref, za_ref, zb_ref):
    agg = s_ref[0] + s_ref[1] + ap_ref[...]
    z = dinv_ref[...] * lax.dot(
        agg, w_ref[...], precision=HIGHEST,
        preferred_element_type=jnp.float32) + b_ref[...]
    zh = z.astype(jnp.bfloat16)
    zl = (z - zh.astype(jnp.float32)).astype(jnp.bfloat16)
    za_ref[...] = zh
    zb_ref[...] = zl


RBZ = 2000


def _zfin(s, ap, b, dinv, W):
    return pl.pallas_call(
        _zfin_body,
        grid=(N // RBZ,),
        in_specs=[
            pl.BlockSpec((NC, RBZ, H2), lambda i: (0, i, 0)),
            pl.BlockSpec((RBZ, H2), lambda i: (i, 0)),
            pl.BlockSpec((1, LATENT), lambda i: (0, 0)),
            pl.BlockSpec((RBZ, 1), lambda i: (i, 0)),
            pl.BlockSpec((H2, LATENT), lambda i: (0, 0)),
        ],
        out_specs=[
            pl.BlockSpec((RBZ, LATENT), lambda i: (i, 0)),
            pl.BlockSpec((RBZ, LATENT), lambda i: (i, 0)),
        ],
        out_shape=[
            jax.ShapeDtypeStruct((N, LATENT), jnp.bfloat16),
            jax.ShapeDtypeStruct((N, LATENT), jnp.bfloat16),
        ],
    )(s, ap, b.reshape(1, LATENT), dinv, W)


def _dec_body(za_ref, zt_ref, o_ref):
    p = lax.dot(za_ref[...], zt_ref[...],
                preferred_element_type=jnp.float32)
    o_ref[...] = jax.nn.sigmoid(p)


def _dec_f32_body(za_ref, zt_ref, o_ref):
    p = lax.dot(za_ref[...], zt_ref[...], precision=HIGHEST,
                preferred_element_type=jnp.float32)
    o_ref[...] = jax.nn.sigmoid(p)


def _dec_f32(z, zT):
    return pl.pallas_call(
        _dec_f32_body,
        grid=(N // RBZ, pl.cdiv(N, CB)),
        in_specs=[
            pl.BlockSpec((RBZ, LATENT), lambda i, j: (i, 0)),
            pl.BlockSpec((LATENT, CB), lambda i, j: (0, j)),
        ],
        out_specs=pl.BlockSpec((RBZ, CB), lambda i, j: (i, j)),
        out_shape=jax.ShapeDtypeStruct((N, N), jnp.float32),
    )(z, zT)


CB = 2048


def _dec(za, zbT):
    return pl.pallas_call(
        _dec_body,
        grid=(N // RBZ, pl.cdiv(N, CB)),
        in_specs=[
            pl.BlockSpec((RBZ, LS), lambda i, j: (i, 0)),
            pl.BlockSpec((LS, CB), lambda i, j: (0, j)),
        ],
        out_specs=pl.BlockSpec((RBZ, CB), lambda i, j: (i, j)),
        out_shape=jax.ShapeDtypeStruct((N, N), jnp.float32),
    )(za, zbT)


def kernel(X, A, W1, b1, W2, b2, Wmu, bmu):
    A32 = A.astype(jnp.int32)
    src = A32[0].reshape(NC, NS, NB, K)
    dst = A32[1].reshape(NC, NS, NB, K)

    cnt = _deg_call(dst)
    dinv, Xp = _pre(cnt, X)

    s1 = _scat128(Xp, src, dst)
    h2p = _l1(s1, Xp, W1, b1, W2, dinv)
    s2 = _scat128(h2p, src, dst)

    a2p = _l2(s2, h2p, b2, dinv)
    s3 = _scat128(a2p, src, dst)
    zh, zl = _zfin(s3, a2p, bmu, dinv, Wmu)

    za = jnp.concatenate([zh, zl, zh], axis=1)
    zbT = jnp.concatenate([zh, zh, zl], axis=1).T
    return _dec(za, zbT)


# speedup vs baseline: 14.9829x; 1.0219x over previous
import functools

import jax
import jax.numpy as jnp
from jax import lax
from jax.experimental import pallas as pl
from jax.experimental.pallas import tpu as pltpu
from jax.experimental.pallas import tpu_sc as plsc

N = 10000
E = 160000
IN_CH = 128
H1 = 256
H2 = 128
LATENT = 64

NC = 2
NS = 16
K = 125
NB = E // (NC * NS * K)
SLAB = 624
TAIL = N - NS * SLAB
ZR = 104
LANES = 16

HIGHEST = lax.Precision.HIGHEST


def _fill(ref, nrows, ncol, value):
    def body(r, _):
        for f in range(ncol // LANES):
            ref[r, pl.ds(f * LANES, LANES)] = jnp.full(
                (LANES,), value, jnp.float32)
        return 0
    lax.fori_loop(0, nrows, body, 0)


def _make_deg():
    mesh = plsc.VectorSubcoreMesh(core_axis_name="c", subcore_axis_name="s")

    @functools.partial(
        pl.kernel,
        out_type=jax.ShapeDtypeStruct((NC, N, LANES), jnp.float32),
        mesh=mesh,
        scratch_types=[
            pltpu.VMEM((NB, K), jnp.int32),
            pltpu.VMEM((K, LANES), jnp.float32),
            pltpu.VMEM((ZR, LANES), jnp.float32),
            pltpu.VMEM_SHARED((N, LANES), jnp.float32),
        ],
    )
    def deg(dst_hbm, cnt_hbm, dst_v, ones_v, zeros_v, acc_sh):
        c = lax.axis_index("c")
        s = lax.axis_index("s")
        pltpu.sync_copy(dst_hbm.at[c, s], dst_v)
        _fill(ones_v, K, LANES, 1.0)
        _fill(zeros_v, ZR, LANES, 0.0)
        row0 = pl.multiple_of(s * SLAB, 8)
        for r in range(SLAB // ZR):
            pltpu.sync_copy(zeros_v,
                            acc_sh.at[pl.ds(pl.multiple_of(row0 + r * ZR, 8),
                                            ZR)])
        @pl.when(s == 0)
        def _():
            pltpu.sync_copy(zeros_v.at[pl.ds(0, TAIL)],
                            acc_sh.at[pl.ds(NS * SLAB, TAIL)])
        plsc.subcore_barrier()
        def body(j, _):
            pltpu.sync_copy(ones_v, acc_sh.at[dst_v.at[j]], add=True)
            return 0
        lax.fori_loop(0, NB, body, 0)
        plsc.subcore_barrier()
        pltpu.sync_copy(acc_sh.at[pl.ds(row0, SLAB)],
                        cnt_hbm.at[c, pl.ds(row0, SLAB)])
        @pl.when(s == 0)
        def _():
            pltpu.sync_copy(acc_sh.at[pl.ds(NS * SLAB, TAIL)],
                            cnt_hbm.at[c, pl.ds(NS * SLAB, TAIL)])

    return deg


def _make_scatter(F):
    mesh = plsc.VectorSubcoreMesh(core_axis_name="c", subcore_axis_name="s")

    @functools.partial(
        pl.kernel,
        out_type=jax.ShapeDtypeStruct((NC, N, F), jnp.float32),
        mesh=mesh,
        scratch_types=[
            pltpu.VMEM((NB, K), jnp.int32),
            pltpu.VMEM((NB, K), jnp.int32),
            pltpu.VMEM((2, K, F), jnp.float32),
            pltpu.VMEM_SHARED((N, F), jnp.float32),
            pltpu.SemaphoreType.DMA,
            pltpu.SemaphoreType.DMA,
        ],
    )
    def scat(table_hbm, src_hbm, dst_hbm, out_hbm,
             src_v, dst_v, rows_v, acc_sh, sem0, sem1):
        c = lax.axis_index("c")
        s = lax.axis_index("s")
        pltpu.sync_copy(src_hbm.at[c, s], src_v)
        pltpu.sync_copy(dst_hbm.at[c, s], dst_v)
        _fill(rows_v.at[0], ZR, F, 0.0)
        row0 = pl.multiple_of(s * SLAB, 8)
        for r in range(SLAB // ZR):
            pltpu.sync_copy(rows_v.at[0, pl.ds(0, ZR)],
                            acc_sh.at[pl.ds(pl.multiple_of(row0 + r * ZR, 8),
                                            ZR)])
        @pl.when(s == 0)
        def _():
            pltpu.sync_copy(rows_v.at[0, pl.ds(0, TAIL)],
                            acc_sh.at[pl.ds(NS * SLAB, TAIL)])
        plsc.subcore_barrier()
        sems = (sem0, sem1)
        desc = [None, None]
        desc[0] = pltpu.async_copy(table_hbm.at[src_v.at[0]],
                                   rows_v.at[0], sem0)
        for j in range(NB):
            b = j % 2
            desc[b].wait()
            if j + 1 < NB:
                nb_ = (j + 1) % 2
                desc[nb_] = pltpu.async_copy(
                    table_hbm.at[src_v.at[j + 1]], rows_v.at[nb_], sems[nb_])
            pltpu.sync_copy(rows_v.at[b], acc_sh.at[dst_v.at[j]], add=True)
        plsc.subcore_barrier()
        pltpu.sync_copy(acc_sh.at[pl.ds(row0, SLAB)],
                        out_hbm.at[c, pl.ds(row0, SLAB)])
        @pl.when(s == 0)
        def _():
            pltpu.sync_copy(acc_sh.at[pl.ds(NS * SLAB, TAIL)],
                            out_hbm.at[c, pl.ds(NS * SLAB, TAIL)])

    return scat


_deg_call = _make_deg()
_scat128 = _make_scatter(128)


RB = 1000


def _pre_body(cnt_ref, x_ref, dinv_ref, xp_ref):
    deg = 1.0 + cnt_ref[0, :, 0:1] + cnt_ref[1, :, 0:1]
    dinv = lax.rsqrt(deg)
    dinv_ref[...] = dinv
    xp_ref[...] = dinv * x_ref[...]


def _pre(cnt, X):
    return pl.pallas_call(
        _pre_body,
        grid=(N // RB,),
        in_specs=[
            pl.BlockSpec((NC, RB, LANES), lambda i: (0, i, 0)),
            pl.BlockSpec((RB, IN_CH), lambda i: (i, 0)),
        ],
        out_specs=[
            pl.BlockSpec((RB, 1), lambda i: (i, 0)),
            pl.BlockSpec((RB, IN_CH), lambda i: (i, 0)),
        ],
        out_shape=[
            jax.ShapeDtypeStruct((N, 1), jnp.float32),
            jax.ShapeDtypeStruct((N, IN_CH), jnp.float32),
        ],
    )(cnt, X)


def _l1_body(s_ref, xp_ref, w1_ref, b1_ref, w2_ref, dinv_ref, o_ref):
    dinv = dinv_ref[...]
    agg = s_ref[0] + s_ref[1] + xp_ref[...]
    u = dinv * lax.dot(agg, w1_ref[...], precision=HIGHEST,
                       preferred_element_type=jnp.float32) + b1_ref[...]
    a = jnp.where(u >= 0, u, 0.2 * u)
    o_ref[...] = dinv * lax.dot(a, w2_ref[...], precision=HIGHEST,
                                preferred_element_type=jnp.float32)


def _l1(s1, Xp, W1, b1, W2, dinv):
    return pl.pallas_call(
        _l1_body,
        grid=(N // RB,),
        in_specs=[
            pl.BlockSpec((NC, RB, IN_CH), lambda i: (0, i, 0)),
            pl.BlockSpec((RB, IN_CH), lambda i: (i, 0)),
            pl.BlockSpec((IN_CH, H1), lambda i: (0, 0)),
            pl.BlockSpec((1, H1), lambda i: (0, 0)),
            pl.BlockSpec((H1, H2), lambda i: (0, 0)),
            pl.BlockSpec((RB, 1), lambda i: (i, 0)),
        ],
        out_specs=pl.BlockSpec((RB, H2), lambda i: (i, 0)),
        out_shape=jax.ShapeDtypeStruct((N, H2), jnp.float32),
    )(s1, Xp, W1, b1.reshape(1, H1), W2, dinv)


def _l2_body(s_ref, hp_ref, b_ref, dinv_ref, o_ref):
    dinv = dinv_ref[...]
    u = dinv * (s_ref[0] + s_ref[1] + hp_ref[...]) + b_ref[...]
    a = jnp.where(u >= 0, u, 0.2 * u)
    o_ref[...] = dinv * a


def _l2(s, hp, b, dinv):
    return pl.pallas_call(
        _l2_body,
        grid=(N // RB,),
        in_specs=[
            pl.BlockSpec((NC, RB, H2), lambda i: (0, i, 0)),
            pl.BlockSpec((RB, H2), lambda i: (i, 0)),
            pl.BlockSpec((1, H2), lambda i: (0, 0)),
            pl.BlockSpec((RB, 1), lambda i: (i, 0)),
        ],
        out_specs=pl.BlockSpec((RB, H2), lambda i: (i, 0)),
        out_shape=jax.ShapeDtypeStruct((N, H2), jnp.float32),
    )(s, hp, b.reshape(1, H2), dinv)


LS = 3 * LATENT


def _zfin_body(s_ref, ap_ref, b_ref, dinv_ref, w_ref, za_ref, zb_ref):
    agg = s_ref[0] + s_ref[1] + ap_ref[...]
    z = dinv_ref[...] * lax.dot(
        agg, w_ref[...], precision=HIGHEST,
        preferred_element_type=jnp.float32) + b_ref[...]
    zh = z.astype(jnp.bfloat16)
    zl = (z - zh.astype(jnp.float32)).astype(jnp.bfloat16)
    za_ref[...] = zh
    zb_ref[...] = zl


RBZ = 2000


def _zfin(s, ap, b, dinv, W):
    return pl.pallas_call(
        _zfin_body,
        grid=(N // RBZ,),
        in_specs=[
            pl.BlockSpec((NC, RBZ, H2), lambda i: (0, i, 0)),
            pl.BlockSpec((RBZ, H2), lambda i: (i, 0)),
            pl.BlockSpec((1, LATENT), lambda i: (0, 0)),
            pl.BlockSpec((RBZ, 1), lambda i: (i, 0)),
            pl.BlockSpec((H2, LATENT), lambda i: (0, 0)),
        ],
        out_specs=[
            pl.BlockSpec((RBZ, LATENT), lambda i: (i, 0)),
            pl.BlockSpec((RBZ, LATENT), lambda i: (i, 0)),
        ],
        out_shape=[
            jax.ShapeDtypeStruct((N, LATENT), jnp.bfloat16),
            jax.ShapeDtypeStruct((N, LATENT), jnp.bfloat16),
        ],
    )(s, ap, b.reshape(1, LATENT), dinv, W)


def _dec_body(za_ref, zt_ref, o_ref):
    p = lax.dot(za_ref[...], zt_ref[...],
                preferred_element_type=jnp.float32)
    pc = jnp.clip(p, -40.0, 40.0)
    o_ref[...] = 0.5 * (1.0 + jnp.tanh(0.5 * pc))


def _dec_f32_body(za_ref, zt_ref, o_ref):
    p = lax.dot(za_ref[...], zt_ref[...], precision=HIGHEST,
                preferred_element_type=jnp.float32)
    o_ref[...] = jax.nn.sigmoid(p)


def _dec_f32(z, zT):
    return pl.pallas_call(
        _dec_f32_body,
        grid=(N // RBZ, pl.cdiv(N, CB)),
        in_specs=[
            pl.BlockSpec((RBZ, LATENT), lambda i, j: (i, 0)),
            pl.BlockSpec((LATENT, CB), lambda i, j: (0, j)),
        ],
        out_specs=pl.BlockSpec((RBZ, CB), lambda i, j: (i, j)),
        out_shape=jax.ShapeDtypeStruct((N, N), jnp.float32),
    )(z, zT)


CB = 2048


def _dec(za, zbT):
    return pl.pallas_call(
        _dec_body,
        grid=(N // RBZ, pl.cdiv(N, CB)),
        in_specs=[
            pl.BlockSpec((RBZ, LS), lambda i, j: (i, 0)),
            pl.BlockSpec((LS, CB), lambda i, j: (0, j)),
        ],
        out_specs=pl.BlockSpec((RBZ, CB), lambda i, j: (i, j)),
        out_shape=jax.ShapeDtypeStruct((N, N), jnp.float32),
    )(za, zbT)


def kernel(X, A, W1, b1, W2, b2, Wmu, bmu):
    A32 = A.astype(jnp.int32)
    src = A32[0].reshape(NC, NS, NB, K)
    dst = A32[1].reshape(NC, NS, NB, K)

    cnt = _deg_call(dst)
    dinv, Xp = _pre(cnt, X)

    s1 = _scat128(Xp, src, dst)
    h2p = _l1(s1, Xp, W1, b1, W2, dinv)
    s2 = _scat128(h2p, src, dst)

    a2p = _l2(s2, h2p, b2, dinv)
    s3 = _scat128(a2p, src, dst)
    zh, zl = _zfin(s3, a2p, bmu, dinv, Wmu)

    za = jnp.concatenate([zh, zl, zh], axis=1)
    zbT = jnp.concatenate([zh, zh, zl], axis=1).T
    return _dec(za, zbT)


# speedup vs baseline: 15.5420x; 1.0373x over previous
import functools

import jax
import jax.numpy as jnp
from jax import lax
from jax.experimental import pallas as pl
from jax.experimental.pallas import tpu as pltpu
from jax.experimental.pallas import tpu_sc as plsc

N = 10000
E = 160000
IN_CH = 128
H1 = 256
H2 = 128
LATENT = 64

NC = 2
NS = 16
K = 125
NB = E // (NC * NS * K)
SLAB = 624
TAIL = N - NS * SLAB
ZR = 104
LANES = 16

HIGHEST = lax.Precision.HIGHEST


def _fill(ref, nrows, ncol, value):
    def body(r, _):
        for f in range(ncol // LANES):
            ref[r, pl.ds(f * LANES, LANES)] = jnp.full(
                (LANES,), value, jnp.float32)
        return 0
    lax.fori_loop(0, nrows, body, 0)


def _make_deg():
    mesh = plsc.VectorSubcoreMesh(core_axis_name="c", subcore_axis_name="s")

    @functools.partial(
        pl.kernel,
        out_type=jax.ShapeDtypeStruct((NC, N, LANES), jnp.float32),
        mesh=mesh,
        scratch_types=[
            pltpu.VMEM((NB, K), jnp.int32),
            pltpu.VMEM((K, LANES), jnp.float32),
            pltpu.VMEM((ZR, LANES), jnp.float32),
            pltpu.VMEM_SHARED((N, LANES), jnp.float32),
        ],
    )
    def deg(dst_hbm, cnt_hbm, dst_v, ones_v, zeros_v, acc_sh):
        c = lax.axis_index("c")
        s = lax.axis_index("s")
        pltpu.sync_copy(dst_hbm.at[c, s], dst_v)
        _fill(ones_v, K, LANES, 1.0)
        _fill(zeros_v, ZR, LANES, 0.0)
        row0 = pl.multiple_of(s * SLAB, 8)
        for r in range(SLAB // ZR):
            pltpu.sync_copy(zeros_v,
                            acc_sh.at[pl.ds(pl.multiple_of(row0 + r * ZR, 8),
                                            ZR)])
        @pl.when(s == 0)
        def _():
            pltpu.sync_copy(zeros_v.at[pl.ds(0, TAIL)],
                            acc_sh.at[pl.ds(NS * SLAB, TAIL)])
        plsc.subcore_barrier()
        def body(j, _):
            pltpu.sync_copy(ones_v, acc_sh.at[dst_v.at[j]], add=True)
            return 0
        lax.fori_loop(0, NB, body, 0)
        plsc.subcore_barrier()
        pltpu.sync_copy(acc_sh.at[pl.ds(row0, SLAB)],
                        cnt_hbm.at[c, pl.ds(row0, SLAB)])
        @pl.when(s == 0)
        def _():
            pltpu.sync_copy(acc_sh.at[pl.ds(NS * SLAB, TAIL)],
                            cnt_hbm.at[c, pl.ds(NS * SLAB, TAIL)])

    return deg


def _make_scatter(F):
    mesh = plsc.VectorSubcoreMesh(core_axis_name="c", subcore_axis_name="s")

    @functools.partial(
        pl.kernel,
        out_type=jax.ShapeDtypeStruct((NC, N, F), jnp.float32),
        mesh=mesh,
        scratch_types=[
            pltpu.VMEM((NB, K), jnp.int32),
            pltpu.VMEM((NB, K), jnp.int32),
            pltpu.VMEM((2, K, F), jnp.float32),
            pltpu.VMEM_SHARED((N, F), jnp.float32),
            pltpu.SemaphoreType.DMA,
            pltpu.SemaphoreType.DMA,
        ],
    )
    def scat(table_hbm, src_hbm, dst_hbm, out_hbm,
             src_v, dst_v, rows_v, acc_sh, sem0, sem1):
        c = lax.axis_index("c")
        s = lax.axis_index("s")
        pltpu.sync_copy(src_hbm.at[c, s], src_v)
        pltpu.sync_copy(dst_hbm.at[c, s], dst_v)
        _fill(rows_v.at[0], ZR, F, 0.0)
        row0 = pl.multiple_of(s * SLAB, 8)
        for r in range(SLAB // ZR):
            pltpu.sync_copy(rows_v.at[0, pl.ds(0, ZR)],
                            acc_sh.at[pl.ds(pl.multiple_of(row0 + r * ZR, 8),
                                            ZR)])
        @pl.when(s == 0)
        def _():
            pltpu.sync_copy(rows_v.at[0, pl.ds(0, TAIL)],
                            acc_sh.at[pl.ds(NS * SLAB, TAIL)])
        plsc.subcore_barrier()
        sems = (sem0, sem1)
        desc = [None, None]
        desc[0] = pltpu.async_copy(table_hbm.at[src_v.at[0]],
                                   rows_v.at[0], sem0)
        for j in range(NB):
            b = j % 2
            desc[b].wait()
            if j + 1 < NB:
                nb_ = (j + 1) % 2
                desc[nb_] = pltpu.async_copy(
                    table_hbm.at[src_v.at[j + 1]], rows_v.at[nb_], sems[nb_])
            pltpu.sync_copy(rows_v.at[b], acc_sh.at[dst_v.at[j]], add=True)
        plsc.subcore_barrier()
        pltpu.sync_copy(acc_sh.at[pl.ds(row0, SLAB)],
                        out_hbm.at[c, pl.ds(row0, SLAB)])
        @pl.when(s == 0)
        def _():
            pltpu.sync_copy(acc_sh.at[pl.ds(NS * SLAB, TAIL)],
                            out_hbm.at[c, pl.ds(NS * SLAB, TAIL)])

    return scat


_deg_call = _make_deg()
_scat128 = _make_scatter(128)


RB = 1000


def _pre_body(cnt_ref, x_ref, dinv_ref, xp_ref):
    deg = 1.0 + cnt_ref[0, :, 0:1] + cnt_ref[1, :, 0:1]
    dinv = lax.rsqrt(deg)
    dinv_ref[...] = dinv
    xp_ref[...] = dinv * x_ref[...]


def _pre(cnt, X):
    return pl.pallas_call(
        _pre_body,
        grid=(N // RB,),
        in_specs=[
            pl.BlockSpec((NC, RB, LANES), lambda i: (0, i, 0)),
            pl.BlockSpec((RB, IN_CH), lambda i: (i, 0)),
        ],
        out_specs=[
            pl.BlockSpec((RB, 1), lambda i: (i, 0)),
            pl.BlockSpec((RB, IN_CH), lambda i: (i, 0)),
        ],
        out_shape=[
            jax.ShapeDtypeStruct((N, 1), jnp.float32),
            jax.ShapeDtypeStruct((N, IN_CH), jnp.float32),
        ],
    )(cnt, X)


def _split_w(W):
    wh = W.astype(jnp.bfloat16)
    wl = (W - wh.astype(jnp.float32)).astype(jnp.bfloat16)
    return wh, wl


def _dot3(x, wh, wl):
    xh = x.astype(jnp.bfloat16)
    xl = (x - xh.astype(jnp.float32)).astype(jnp.bfloat16)
    acc = lax.dot(xh, wh, preferred_element_type=jnp.float32)
    acc = acc + lax.dot(xl, wh, preferred_element_type=jnp.float32)
    return acc + lax.dot(xh, wl, preferred_element_type=jnp.float32)


def _l1_body(s_ref, xp_ref, w1h_ref, w1l_ref, b1_ref, w2h_ref, w2l_ref,
             dinv_ref, o_ref):
    dinv = dinv_ref[...]
    agg = s_ref[0] + s_ref[1] + xp_ref[...]
    u = dinv * _dot3(agg, w1h_ref[...], w1l_ref[...]) + b1_ref[...]
    a = jnp.where(u >= 0, u, 0.2 * u)
    o_ref[...] = dinv * _dot3(a, w2h_ref[...], w2l_ref[...])


def _l1(s1, Xp, W1, b1, W2, dinv):
    w1h, w1l = _split_w(W1)
    w2h, w2l = _split_w(W2)
    return pl.pallas_call(
        _l1_body,
        grid=(N // RB,),
        in_specs=[
            pl.BlockSpec((NC, RB, IN_CH), lambda i: (0, i, 0)),
            pl.BlockSpec((RB, IN_CH), lambda i: (i, 0)),
            pl.BlockSpec((IN_CH, H1), lambda i: (0, 0)),
            pl.BlockSpec((IN_CH, H1), lambda i: (0, 0)),
            pl.BlockSpec((1, H1), lambda i: (0, 0)),
            pl.BlockSpec((H1, H2), lambda i: (0, 0)),
            pl.BlockSpec((H1, H2), lambda i: (0, 0)),
            pl.BlockSpec((RB, 1), lambda i: (i, 0)),
        ],
        out_specs=pl.BlockSpec((RB, H2), lambda i: (i, 0)),
        out_shape=jax.ShapeDtypeStruct((N, H2), jnp.float32),
    )(s1, Xp, w1h, w1l, b1.reshape(1, H1), w2h, w2l, dinv)


def _l2_body(s_ref, hp_ref, b_ref, dinv_ref, o_ref):
    dinv = dinv_ref[...]
    u = dinv * (s_ref[0] + s_ref[1] + hp_ref[...]) + b_ref[...]
    a = jnp.where(u >= 0, u, 0.2 * u)
    o_ref[...] = dinv * a


def _l2(s, hp, b, dinv):
    return pl.pallas_call(
        _l2_body,
        grid=(N // RB,),
        in_specs=[
            pl.BlockSpec((NC, RB, H2), lambda i: (0, i, 0)),
            pl.BlockSpec((RB, H2), lambda i: (i, 0)),
            pl.BlockSpec((1, H2), lambda i: (0, 0)),
            pl.BlockSpec((RB, 1), lambda i: (i, 0)),
        ],
        out_specs=pl.BlockSpec((RB, H2), lambda i: (i, 0)),
        out_shape=jax.ShapeDtypeStruct((N, H2), jnp.float32),
    )(s, hp, b.reshape(1, H2), dinv)


LS = 3 * LATENT


def _zfin_body(s_ref, ap_ref, b_ref, dinv_ref, wh_ref, wl_ref,
               za_ref, zb_ref):
    agg = s_ref[0] + s_ref[1] + ap_ref[...]
    z = (dinv_ref[...] * _dot3(agg, wh_ref[...], wl_ref[...])
         + b_ref[...])
    zh = z.astype(jnp.bfloat16)
    zl = (z - zh.astype(jnp.float32)).astype(jnp.bfloat16)
    za_ref[...] = zh
    zb_ref[...] = zl


RBZ = 2000


def _zfin(s, ap, b, dinv, W):
    wh, wl = _split_w(W)
    return pl.pallas_call(
        _zfin_body,
        grid=(N // RBZ,),
        in_specs=[
            pl.BlockSpec((NC, RBZ, H2), lambda i: (0, i, 0)),
            pl.BlockSpec((RBZ, H2), lambda i: (i, 0)),
            pl.BlockSpec((1, LATENT), lambda i: (0, 0)),
            pl.BlockSpec((RBZ, 1), lambda i: (i, 0)),
            pl.BlockSpec((H2, LATENT), lambda i: (0, 0)),
            pl.BlockSpec((H2, LATENT), lambda i: (0, 0)),
        ],
        out_specs=[
            pl.BlockSpec((RBZ, LATENT), lambda i: (i, 0)),
            pl.BlockSpec((RBZ, LATENT), lambda i: (i, 0)),
        ],
        out_shape=[
            jax.ShapeDtypeStruct((N, LATENT), jnp.bfloat16),
            jax.ShapeDtypeStruct((N, LATENT), jnp.bfloat16),
        ],
    )(s, ap, b.reshape(1, LATENT), dinv, wh, wl)


def _dec_body(za_ref, zt_ref, o_ref):
    p = lax.dot(za_ref[...], zt_ref[...],
                preferred_element_type=jnp.float32)
    pc = jnp.clip(p, -40.0, 40.0)
    o_ref[...] = 0.5 * (1.0 + jnp.tanh(0.5 * pc))


def _dec_f32_body(za_ref, zt_ref, o_ref):
    p = lax.dot(za_ref[...], zt_ref[...], precision=HIGHEST,
                preferred_element_type=jnp.float32)
    o_ref[...] = jax.nn.sigmoid(p)


def _dec_f32(z, zT):
    return pl.pallas_call(
        _dec_f32_body,
        grid=(N // RBZ, pl.cdiv(N, CB)),
        in_specs=[
            pl.BlockSpec((RBZ, LATENT), lambda i, j: (i, 0)),
            pl.BlockSpec((LATENT, CB), lambda i, j: (0, j)),
        ],
        out_specs=pl.BlockSpec((RBZ, CB), lambda i, j: (i, j)),
        out_shape=jax.ShapeDtypeStruct((N, N), jnp.float32),
    )(z, zT)


CB = 2048


def _dec(za, zbT):
    return pl.pallas_call(
        _dec_body,
        grid=(N // RBZ, pl.cdiv(N, CB)),
        in_specs=[
            pl.BlockSpec((RBZ, LS), lambda i, j: (i, 0)),
            pl.BlockSpec((LS, CB), lambda i, j: (0, j)),
        ],
        out_specs=pl.BlockSpec((RBZ, CB), lambda i, j: (i, j)),
        out_shape=jax.ShapeDtypeStruct((N, N), jnp.float32),
    )(za, zbT)


def kernel(X, A, W1, b1, W2, b2, Wmu, bmu):
    A32 = A.astype(jnp.int32)
    src = A32[0].reshape(NC, NS, NB, K)
    dst = A32[1].reshape(NC, NS, NB, K)

    cnt = _deg_call(dst)
    dinv, Xp = _pre(cnt, X)

    s1 = _scat128(Xp, src, dst)
    h2p = _l1(s1, Xp, W1, b1, W2, dinv)
    s2 = _scat128(h2p, src, dst)

    a2p = _l2(s2, h2p, b2, dinv)
    s3 = _scat128(a2p, src, dst)
    zh, zl = _zfin(s3, a2p, bmu, dinv, Wmu)

    za = jnp.concatenate([zh, zl, zh], axis=1)
    zbT = jnp.concatenate([zh, zh, zl], axis=1).T
    return _dec(za, zbT)


# speedup vs baseline: 15.5674x; 1.0016x over previous
import functools

import jax
import jax.numpy as jnp
from jax import lax
from jax.experimental import pallas as pl
from jax.experimental.pallas import tpu as pltpu
from jax.experimental.pallas import tpu_sc as plsc

N = 10000
E = 160000
IN_CH = 128
H1 = 256
H2 = 128
LATENT = 64

NC = 2
NS = 16
K = 125
NB = E // (NC * NS * K)
SLAB = 624
TAIL = N - NS * SLAB
ZR = 104
LANES = 16

HIGHEST = lax.Precision.HIGHEST


def _fill(ref, nrows, ncol, value):
    def body(r, _):
        for f in range(ncol // LANES):
            ref[r, pl.ds(f * LANES, LANES)] = jnp.full(
                (LANES,), value, jnp.float32)
        return 0
    lax.fori_loop(0, nrows, body, 0)


def _make_deg():
    mesh = plsc.VectorSubcoreMesh(core_axis_name="c", subcore_axis_name="s")

    @functools.partial(
        pl.kernel,
        out_type=jax.ShapeDtypeStruct((NC, N, LANES), jnp.float32),
        mesh=mesh,
        scratch_types=[
            pltpu.VMEM((NB, K), jnp.int32),
            pltpu.VMEM((K, LANES), jnp.float32),
            pltpu.VMEM((ZR, LANES), jnp.float32),
            pltpu.VMEM_SHARED((N, LANES), jnp.float32),
            pltpu.SemaphoreType.DMA,
        ],
    )
    def deg(dst_hbm, cnt_hbm, dst_v, ones_v, zeros_v, acc_sh, sem):
        c = lax.axis_index("c")
        s = lax.axis_index("s")
        pltpu.sync_copy(dst_hbm.at[c, s], dst_v)
        _fill(ones_v, K, LANES, 1.0)
        _fill(zeros_v, ZR, LANES, 0.0)
        row0 = pl.multiple_of(s * SLAB, 8)
        for r in range(SLAB // ZR):
            pltpu.sync_copy(zeros_v,
                            acc_sh.at[pl.ds(pl.multiple_of(row0 + r * ZR, 8),
                                            ZR)])
        @pl.when(s == 0)
        def _():
            pltpu.sync_copy(zeros_v.at[pl.ds(0, TAIL)],
                            acc_sh.at[pl.ds(NS * SLAB, TAIL)])
        plsc.subcore_barrier()
        descs = [pltpu.async_copy(ones_v, acc_sh.at[dst_v.at[j]], sem,
                                  add=True)
                 for j in range(NB)]
        for d in descs:
            d.wait()
        plsc.subcore_barrier()
        pltpu.sync_copy(acc_sh.at[pl.ds(row0, SLAB)],
                        cnt_hbm.at[c, pl.ds(row0, SLAB)])
        @pl.when(s == 0)
        def _():
            pltpu.sync_copy(acc_sh.at[pl.ds(NS * SLAB, TAIL)],
                            cnt_hbm.at[c, pl.ds(NS * SLAB, TAIL)])

    return deg


def _make_scatter(F):
    mesh = plsc.VectorSubcoreMesh(core_axis_name="c", subcore_axis_name="s")

    @functools.partial(
        pl.kernel,
        out_type=jax.ShapeDtypeStruct((NC, N, F), jnp.float32),
        mesh=mesh,
        scratch_types=[
            pltpu.VMEM((NB, K), jnp.int32),
            pltpu.VMEM((NB, K), jnp.int32),
            pltpu.VMEM((2, K, F), jnp.float32),
            pltpu.VMEM_SHARED((N, F), jnp.float32),
            pltpu.SemaphoreType.DMA,
            pltpu.SemaphoreType.DMA,
            pltpu.SemaphoreType.DMA,
            pltpu.SemaphoreType.DMA,
        ],
    )
    def scat(table_hbm, src_hbm, dst_hbm, out_hbm,
             src_v, dst_v, rows_v, acc_sh, gsem0, gsem1, ssem0, ssem1):
        c = lax.axis_index("c")
        s = lax.axis_index("s")
        pltpu.sync_copy(src_hbm.at[c, s], src_v)
        pltpu.sync_copy(dst_hbm.at[c, s], dst_v)
        _fill(rows_v.at[0], ZR, F, 0.0)
        row0 = pl.multiple_of(s * SLAB, 8)
        for r in range(SLAB // ZR):
            pltpu.sync_copy(rows_v.at[0, pl.ds(0, ZR)],
                            acc_sh.at[pl.ds(pl.multiple_of(row0 + r * ZR, 8),
                                            ZR)])
        @pl.when(s == 0)
        def _():
            pltpu.sync_copy(rows_v.at[0, pl.ds(0, TAIL)],
                            acc_sh.at[pl.ds(NS * SLAB, TAIL)])
        plsc.subcore_barrier()
        gsems = (gsem0, gsem1)
        ssems = (ssem0, ssem1)
        gd = [None, None]
        sd = [None, None]
        gd[0] = pltpu.async_copy(table_hbm.at[src_v.at[0]],
                                 rows_v.at[0], gsem0)
        for j in range(NB):
            b = j % 2
            o = (j + 1) % 2
            gd[b].wait()
            if j >= 1:
                sd[o].wait()
            if j + 1 < NB:
                gd[o] = pltpu.async_copy(
                    table_hbm.at[src_v.at[j + 1]], rows_v.at[o], gsems[o])
            sd[b] = pltpu.async_copy(rows_v.at[b], acc_sh.at[dst_v.at[j]],
                                     ssems[b], add=True)
        sd[(NB - 1) % 2].wait()
        plsc.subcore_barrier()
        pltpu.sync_copy(acc_sh.at[pl.ds(row0, SLAB)],
                        out_hbm.at[c, pl.ds(row0, SLAB)])
        @pl.when(s == 0)
        def _():
            pltpu.sync_copy(acc_sh.at[pl.ds(NS * SLAB, TAIL)],
                            out_hbm.at[c, pl.ds(NS * SLAB, TAIL)])

    return scat


_deg_call = _make_deg()
_scat128 = _make_scatter(128)


RB = 1000


def _pre_body(cnt_ref, x_ref, dinv_ref, xp_ref):
    deg = 1.0 + cnt_ref[0, :, 0:1] + cnt_ref[1, :, 0:1]
    dinv = lax.rsqrt(deg)
    dinv_ref[...] = dinv
    xp_ref[...] = dinv * x_ref[...]


def _pre(cnt, X):
    return pl.pallas_call(
        _pre_body,
        grid=(N // RB,),
        in_specs=[
            pl.BlockSpec((NC, RB, LANES), lambda i: (0, i, 0)),
            pl.BlockSpec((RB, IN_CH), lambda i: (i, 0)),
        ],
        out_specs=[
            pl.BlockSpec((RB, 1), lambda i: (i, 0)),
            pl.BlockSpec((RB, IN_CH), lambda i: (i, 0)),
        ],
        out_shape=[
            jax.ShapeDtypeStruct((N, 1), jnp.float32),
            jax.ShapeDtypeStruct((N, IN_CH), jnp.float32),
        ],
    )(cnt, X)


def _split_w(W):
    wh = W.astype(jnp.bfloat16)
    wl = (W - wh.astype(jnp.float32)).astype(jnp.bfloat16)
    return wh, wl


def _dot3(x, wh, wl):
    xh = x.astype(jnp.bfloat16)
    xl = (x - xh.astype(jnp.float32)).astype(jnp.bfloat16)
    acc = lax.dot(xh, wh, preferred_element_type=jnp.float32)
    acc = acc + lax.dot(xl, wh, preferred_element_type=jnp.float32)
    return acc + lax.dot(xh, wl, preferred_element_type=jnp.float32)


def _l1_body(s_ref, xp_ref, w1h_ref, w1l_ref, b1_ref, w2h_ref, w2l_ref,
             dinv_ref, o_ref):
    dinv = dinv_ref[...]
    agg = s_ref[0] + s_ref[1] + xp_ref[...]
    u = dinv * _dot3(agg, w1h_ref[...], w1l_ref[...]) + b1_ref[...]
    a = jnp.where(u >= 0, u, 0.2 * u)
    o_ref[...] = dinv * _dot3(a, w2h_ref[...], w2l_ref[...])


def _l1(s1, Xp, W1, b1, W2, dinv):
    w1h, w1l = _split_w(W1)
    w2h, w2l = _split_w(W2)
    return pl.pallas_call(
        _l1_body,
        grid=(N // RB,),
        in_specs=[
            pl.BlockSpec((NC, RB, IN_CH), lambda i: (0, i, 0)),
            pl.BlockSpec((RB, IN_CH), lambda i: (i, 0)),
            pl.BlockSpec((IN_CH, H1), lambda i: (0, 0)),
            pl.BlockSpec((IN_CH, H1), lambda i: (0, 0)),
            pl.BlockSpec((1, H1), lambda i: (0, 0)),
            pl.BlockSpec((H1, H2), lambda i: (0, 0)),
            pl.BlockSpec((H1, H2), lambda i: (0, 0)),
            pl.BlockSpec((RB, 1), lambda i: (i, 0)),
        ],
        out_specs=pl.BlockSpec((RB, H2), lambda i: (i, 0)),
        out_shape=jax.ShapeDtypeStruct((N, H2), jnp.float32),
    )(s1, Xp, w1h, w1l, b1.reshape(1, H1), w2h, w2l, dinv)


def _l2_body(s_ref, hp_ref, b_ref, dinv_ref, o_ref):
    dinv = dinv_ref[...]
    u = dinv * (s_ref[0] + s_ref[1] + hp_ref[...]) + b_ref[...]
    a = jnp.where(u >= 0, u, 0.2 * u)
    o_ref[...] = dinv * a


def _l2(s, hp, b, dinv):
    return pl.pallas_call(
        _l2_body,
        grid=(N // RB,),
        in_specs=[
            pl.BlockSpec((NC, RB, H2), lambda i: (0, i, 0)),
            pl.BlockSpec((RB, H2), lambda i: (i, 0)),
            pl.BlockSpec((1, H2), lambda i: (0, 0)),
            pl.BlockSpec((RB, 1), lambda i: (i, 0)),
        ],
        out_specs=pl.BlockSpec((RB, H2), lambda i: (i, 0)),
        out_shape=jax.ShapeDtypeStruct((N, H2), jnp.float32),
    )(s, hp, b.reshape(1, H2), dinv)


LS = 3 * LATENT


def _zfin_body(s_ref, ap_ref, b_ref, dinv_ref, wh_ref, wl_ref,
               za_ref, zb_ref):
    agg = s_ref[0] + s_ref[1] + ap_ref[...]
    z = (dinv_ref[...] * _dot3(agg, wh_ref[...], wl_ref[...])
         + b_ref[...])
    zh = z.astype(jnp.bfloat16)
    zl = (z - zh.astype(jnp.float32)).astype(jnp.bfloat16)
    za_ref[...] = zh
    zb_ref[...] = zl


RBZ = 2000


def _zfin(s, ap, b, dinv, W):
    wh, wl = _split_w(W)
    return pl.pallas_call(
        _zfin_body,
        grid=(N // RBZ,),
        in_specs=[
            pl.BlockSpec((NC, RBZ, H2), lambda i: (0, i, 0)),
            pl.BlockSpec((RBZ, H2), lambda i: (i, 0)),
            pl.BlockSpec((1, LATENT), lambda i: (0, 0)),
            pl.BlockSpec((RBZ, 1), lambda i: (i, 0)),
            pl.BlockSpec((H2, LATENT), lambda i: (0, 0)),
            pl.BlockSpec((H2, LATENT), lambda i: (0, 0)),
        ],
        out_specs=[
            pl.BlockSpec((RBZ, LATENT), lambda i: (i, 0)),
            pl.BlockSpec((RBZ, LATENT), lambda i: (i, 0)),
        ],
        out_shape=[
            jax.ShapeDtypeStruct((N, LATENT), jnp.bfloat16),
            jax.ShapeDtypeStruct((N, LATENT), jnp.bfloat16),
        ],
    )(s, ap, b.reshape(1, LATENT), dinv, wh, wl)


def _dec_body(za_ref, zt_ref, o_ref):
    p = lax.dot(za_ref[...], zt_ref[...],
                preferred_element_type=jnp.float32)
    pc = jnp.clip(p, -40.0, 40.0)
    o_ref[...] = 0.5 * (1.0 + jnp.tanh(0.5 * pc))


def _dec_f32_body(za_ref, zt_ref, o_ref):
    p = lax.dot(za_ref[...], zt_ref[...], precision=HIGHEST,
                preferred_element_type=jnp.float32)
    o_ref[...] = jax.nn.sigmoid(p)


def _dec_f32(z, zT):
    return pl.pallas_call(
        _dec_f32_body,
        grid=(N // RBZ, pl.cdiv(N, CB)),
        in_specs=[
            pl.BlockSpec((RBZ, LATENT), lambda i, j: (i, 0)),
            pl.BlockSpec((LATENT, CB), lambda i, j: (0, j)),
        ],
        out_specs=pl.BlockSpec((RBZ, CB), lambda i, j: (i, j)),
        out_shape=jax.ShapeDtypeStruct((N, N), jnp.float32),
    )(z, zT)


CB = 2048


def _dec(za, zbT):
    return pl.pallas_call(
        _dec_body,
        grid=(N // RBZ, pl.cdiv(N, CB)),
        in_specs=[
            pl.BlockSpec((RBZ, LS), lambda i, j: (i, 0)),
            pl.BlockSpec((LS, CB), lambda i, j: (0, j)),
        ],
        out_specs=pl.BlockSpec((RBZ, CB), lambda i, j: (i, j)),
        out_shape=jax.ShapeDtypeStruct((N, N), jnp.float32),
    )(za, zbT)


def kernel(X, A, W1, b1, W2, b2, Wmu, bmu):
    A32 = A.astype(jnp.int32)
    src = A32[0].reshape(NC, NS, NB, K)
    dst = A32[1].reshape(NC, NS, NB, K)

    cnt = _deg_call(dst)
    dinv, Xp = _pre(cnt, X)

    s1 = _scat128(Xp, src, dst)
    h2p = _l1(s1, Xp, W1, b1, W2, dinv)
    s2 = _scat128(h2p, src, dst)

    a2p = _l2(s2, h2p, b2, dinv)
    s3 = _scat128(a2p, src, dst)
    zh, zl = _zfin(s3, a2p, bmu, dinv, Wmu)

    za = jnp.concatenate([zh, zl, zh], axis=1)
    zbT = jnp.concatenate([zh, zh, zl], axis=1).T
    return _dec(za, zbT)


# speedup vs baseline: 16.1059x; 1.0346x over previous
import functools

import jax
import jax.numpy as jnp
from jax import lax
from jax.experimental import pallas as pl
from jax.experimental.pallas import tpu as pltpu
from jax.experimental.pallas import tpu_sc as plsc

N = 10000
E = 160000
IN_CH = 128
H1 = 256
H2 = 128
LATENT = 64

NC = 2
NS = 16
K = 125
NB = E // (NC * NS * K)
SLAB = 624
TAIL = N - NS * SLAB
ZR = 104
LANES = 16

HIGHEST = lax.Precision.HIGHEST


def _fill(ref, nrows, ncol, value):
    def body(r, _):
        for f in range(ncol // LANES):
            ref[r, pl.ds(f * LANES, LANES)] = jnp.full(
                (LANES,), value, jnp.float32)
        return 0
    lax.fori_loop(0, nrows, body, 0)


def _make_deg():
    mesh = plsc.VectorSubcoreMesh(core_axis_name="c", subcore_axis_name="s")

    @functools.partial(
        pl.kernel,
        out_type=jax.ShapeDtypeStruct((NC, N, LANES), jnp.float32),
        mesh=mesh,
        scratch_types=[
            pltpu.VMEM((NB, K), jnp.int32),
            pltpu.VMEM((K, LANES), jnp.float32),
            pltpu.VMEM((ZR, LANES), jnp.float32),
            pltpu.VMEM_SHARED((N, LANES), jnp.float32),
            pltpu.SemaphoreType.DMA,
        ],
    )
    def deg(dst_hbm, cnt_hbm, dst_v, ones_v, zeros_v, acc_sh, sem):
        c = lax.axis_index("c")
        s = lax.axis_index("s")
        pltpu.sync_copy(dst_hbm.at[c, s], dst_v)
        _fill(ones_v, K, LANES, 1.0)
        _fill(zeros_v, ZR, LANES, 0.0)
        row0 = pl.multiple_of(s * SLAB, 8)
        for r in range(SLAB // ZR):
            pltpu.sync_copy(zeros_v,
                            acc_sh.at[pl.ds(pl.multiple_of(row0 + r * ZR, 8),
                                            ZR)])
        @pl.when(s == 0)
        def _():
            pltpu.sync_copy(zeros_v.at[pl.ds(0, TAIL)],
                            acc_sh.at[pl.ds(NS * SLAB, TAIL)])
        plsc.subcore_barrier()
        descs = [pltpu.async_copy(ones_v, acc_sh.at[dst_v.at[j]], sem,
                                  add=True)
                 for j in range(NB)]
        for d in descs:
            d.wait()
        plsc.subcore_barrier()
        pltpu.sync_copy(acc_sh.at[pl.ds(row0, SLAB)],
                        cnt_hbm.at[c, pl.ds(row0, SLAB)])
        @pl.when(s == 0)
        def _():
            pltpu.sync_copy(acc_sh.at[pl.ds(NS * SLAB, TAIL)],
                            cnt_hbm.at[c, pl.ds(NS * SLAB, TAIL)])

    return deg


def _make_scatter(F):
    mesh = plsc.VectorSubcoreMesh(core_axis_name="c", subcore_axis_name="s")

    @functools.partial(
        pl.kernel,
        out_type=jax.ShapeDtypeStruct((NC, N, F), jnp.float32),
        mesh=mesh,
        scratch_types=[
            pltpu.VMEM((NB, K), jnp.int32),
            pltpu.VMEM((NB, K), jnp.int32),
            pltpu.VMEM((2, K, F), jnp.float32),
            pltpu.VMEM_SHARED((N, F), jnp.float32),
            pltpu.SemaphoreType.DMA,
            pltpu.SemaphoreType.DMA,
            pltpu.SemaphoreType.DMA,
            pltpu.SemaphoreType.DMA,
        ],
    )
    def scat(table_hbm, src_hbm, dst_hbm, out_hbm,
             src_v, dst_v, rows_v, acc_sh, gsem0, gsem1, ssem0, ssem1):
        c = lax.axis_index("c")
        s = lax.axis_index("s")
        pltpu.sync_copy(src_hbm.at[c, s], src_v)
        pltpu.sync_copy(dst_hbm.at[c, s], dst_v)
        _fill(rows_v.at[0], ZR, F, 0.0)
        row0 = pl.multiple_of(s * SLAB, 8)
        for r in range(SLAB // ZR):
            pltpu.sync_copy(rows_v.at[0, pl.ds(0, ZR)],
                            acc_sh.at[pl.ds(pl.multiple_of(row0 + r * ZR, 8),
                                            ZR)])
        @pl.when(s == 0)
        def _():
            pltpu.sync_copy(rows_v.at[0, pl.ds(0, TAIL)],
                            acc_sh.at[pl.ds(NS * SLAB, TAIL)])
        plsc.subcore_barrier()
        gsems = (gsem0, gsem1)
        ssems = (ssem0, ssem1)
        gd = [None, None]
        sd = [None, None]
        gd[0] = pltpu.async_copy(table_hbm.at[src_v.at[0]],
                                 rows_v.at[0], gsem0)
        for j in range(NB):
            b = j % 2
            o = (j + 1) % 2
            gd[b].wait()
            if j >= 1:
                sd[o].wait()
            if j + 1 < NB:
                gd[o] = pltpu.async_copy(
                    table_hbm.at[src_v.at[j + 1]], rows_v.at[o], gsems[o])
            sd[b] = pltpu.async_copy(rows_v.at[b], acc_sh.at[dst_v.at[j]],
                                     ssems[b], add=True)
        sd[(NB - 1) % 2].wait()
        plsc.subcore_barrier()
        pltpu.sync_copy(acc_sh.at[pl.ds(row0, SLAB)],
                        out_hbm.at[c, pl.ds(row0, SLAB)])
        @pl.when(s == 0)
        def _():
            pltpu.sync_copy(acc_sh.at[pl.ds(NS * SLAB, TAIL)],
                            out_hbm.at[c, pl.ds(NS * SLAB, TAIL)])

    return scat


_deg_call = _make_deg()
_scat128 = _make_scatter(128)


RB = 1000


def _pre_body(cnt_ref, x_ref, dinv_ref, xp_ref):
    deg = 1.0 + cnt_ref[0, :, 0:1] + cnt_ref[1, :, 0:1]
    dinv = lax.rsqrt(deg)
    dinv_ref[...] = dinv
    xp_ref[...] = dinv * x_ref[...]


def _pre(cnt, X):
    return pl.pallas_call(
        _pre_body,
        grid=(N // RB,),
        in_specs=[
            pl.BlockSpec((NC, RB, LANES), lambda i: (0, i, 0)),
            pl.BlockSpec((RB, IN_CH), lambda i: (i, 0)),
        ],
        out_specs=[
            pl.BlockSpec((RB, 1), lambda i: (i, 0)),
            pl.BlockSpec((RB, IN_CH), lambda i: (i, 0)),
        ],
        out_shape=[
            jax.ShapeDtypeStruct((N, 1), jnp.float32),
            jax.ShapeDtypeStruct((N, IN_CH), jnp.float32),
        ],
    )(cnt, X)


def _split_w(W):
    wh = W.astype(jnp.bfloat16)
    wl = (W - wh.astype(jnp.float32)).astype(jnp.bfloat16)
    return wh, wl


def _dot3(x, wh, wl):
    xh = x.astype(jnp.bfloat16)
    xl = (x - xh.astype(jnp.float32)).astype(jnp.bfloat16)
    acc = lax.dot(xh, wh, preferred_element_type=jnp.float32)
    acc = acc + lax.dot(xl, wh, preferred_element_type=jnp.float32)
    return acc + lax.dot(xh, wl, preferred_element_type=jnp.float32)


def _l1_body(s_ref, xp_ref, w1h_ref, w1l_ref, b1_ref, w2h_ref, w2l_ref,
             dinv_ref, o_ref):
    dinv = dinv_ref[...]
    agg = s_ref[0] + s_ref[1] + xp_ref[...]
    u = dinv * _dot3(agg, w1h_ref[...], w1l_ref[...]) + b1_ref[...]
    a = jnp.where(u >= 0, u, 0.2 * u)
    o_ref[...] = dinv * _dot3(a, w2h_ref[...], w2l_ref[...])


def _l1(s1, Xp, W1, b1, W2, dinv):
    w1h, w1l = _split_w(W1)
    w2h, w2l = _split_w(W2)
    return pl.pallas_call(
        _l1_body,
        grid=(N // RB,),
        in_specs=[
            pl.BlockSpec((NC, RB, IN_CH), lambda i: (0, i, 0)),
            pl.BlockSpec((RB, IN_CH), lambda i: (i, 0)),
            pl.BlockSpec((IN_CH, H1), lambda i: (0, 0)),
            pl.BlockSpec((IN_CH, H1), lambda i: (0, 0)),
            pl.BlockSpec((1, H1), lambda i: (0, 0)),
            pl.BlockSpec((H1, H2), lambda i: (0, 0)),
            pl.BlockSpec((H1, H2), lambda i: (0, 0)),
            pl.BlockSpec((RB, 1), lambda i: (i, 0)),
        ],
        out_specs=pl.BlockSpec((RB, H2), lambda i: (i, 0)),
        out_shape=jax.ShapeDtypeStruct((N, H2), jnp.float32),
    )(s1, Xp, w1h, w1l, b1.reshape(1, H1), w2h, w2l, dinv)


def _l2_body(s_ref, hp_ref, b_ref, dinv_ref, o_ref):
    dinv = dinv_ref[...]
    u = dinv * (s_ref[0] + s_ref[1] + hp_ref[...]) + b_ref[...]
    a = jnp.where(u >= 0, u, 0.2 * u)
    o_ref[...] = dinv * a


def _l2(s, hp, b, dinv):
    return pl.pallas_call(
        _l2_body,
        grid=(N // RB,),
        in_specs=[
            pl.BlockSpec((NC, RB, H2), lambda i: (0, i, 0)),
            pl.BlockSpec((RB, H2), lambda i: (i, 0)),
            pl.BlockSpec((1, H2), lambda i: (0, 0)),
            pl.BlockSpec((RB, 1), lambda i: (i, 0)),
        ],
        out_specs=pl.BlockSpec((RB, H2), lambda i: (i, 0)),
        out_shape=jax.ShapeDtypeStruct((N, H2), jnp.float32),
    )(s, hp, b.reshape(1, H2), dinv)


LS = 3 * LATENT


def _zfin_body(s_ref, ap_ref, b_ref, dinv_ref, wh_ref, wl_ref,
               za_ref, zbt_ref):
    agg = s_ref[0] + s_ref[1] + ap_ref[...]
    z = (dinv_ref[...] * _dot3(agg, wh_ref[...], wl_ref[...])
         + b_ref[...])
    zh = z.astype(jnp.bfloat16)
    zl = (z - zh.astype(jnp.float32)).astype(jnp.bfloat16)
    za_ref[...] = jnp.concatenate([zh, zl, zh], axis=1)
    zbt_ref[...] = jnp.concatenate([zh.T, zh.T, zl.T], axis=0)


RBZ = 2048


def _zfin(s, ap, b, dinv, W):
    wh, wl = _split_w(W)
    return pl.pallas_call(
        _zfin_body,
        grid=(pl.cdiv(N, RBZ),),
        in_specs=[
            pl.BlockSpec((NC, RBZ, H2), lambda i: (0, i, 0)),
            pl.BlockSpec((RBZ, H2), lambda i: (i, 0)),
            pl.BlockSpec((1, LATENT), lambda i: (0, 0)),
            pl.BlockSpec((RBZ, 1), lambda i: (i, 0)),
            pl.BlockSpec((H2, LATENT), lambda i: (0, 0)),
            pl.BlockSpec((H2, LATENT), lambda i: (0, 0)),
        ],
        out_specs=[
            pl.BlockSpec((RBZ, LS), lambda i: (i, 0)),
            pl.BlockSpec((LS, RBZ), lambda i: (0, i)),
        ],
        out_shape=[
            jax.ShapeDtypeStruct((N, LS), jnp.bfloat16),
            jax.ShapeDtypeStruct((LS, N), jnp.bfloat16),
        ],
    )(s, ap, b.reshape(1, LATENT), dinv, wh, wl)


def _dec_body(za_ref, zt_ref, o_ref):
    p = lax.dot(za_ref[...], zt_ref[...],
                preferred_element_type=jnp.float32)
    pc = jnp.clip(p, -40.0, 40.0)
    o_ref[...] = 0.5 * (1.0 + jnp.tanh(0.5 * pc))


CB = 2048


def _dec(za, zbT):
    return pl.pallas_call(
        _dec_body,
        grid=(pl.cdiv(N, RBZ), pl.cdiv(N, CB)),
        in_specs=[
            pl.BlockSpec((RBZ, LS), lambda i, j: (i, 0)),
            pl.BlockSpec((LS, CB), lambda i, j: (0, j)),
        ],
        out_specs=pl.BlockSpec((RBZ, CB), lambda i, j: (i, j)),
        out_shape=jax.ShapeDtypeStruct((N, N), jnp.float32),
    )(za, zbT)


def kernel(X, A, W1, b1, W2, b2, Wmu, bmu):
    A32 = A.astype(jnp.int32)
    src = A32[0].reshape(NC, NS, NB, K)
    dst = A32[1].reshape(NC, NS, NB, K)

    cnt = _deg_call(dst)
    dinv, Xp = _pre(cnt, X)

    s1 = _scat128(Xp, src, dst)
    h2p = _l1(s1, Xp, W1, b1, W2, dinv)
    s2 = _scat128(h2p, src, dst)

    a2p = _l2(s2, h2p, b2, dinv)
    s3 = _scat128(a2p, src, dst)
    za, zbT = _zfin(s3, a2p, bmu, dinv, Wmu)

    return _dec(za, zbT)
